# baseline (speedup 1.0000x reference)
import functools

import jax
import jax.numpy as jnp
from jax import lax
from jax.experimental import pallas as pl
from jax.experimental.pallas import tpu as pltpu
from jax.experimental.pallas import tpu_sc as plsc

BATCH = 128
VOCAB = 100000
NC = 2
NS = 16
L = 16
R = 8
NG = BATCH // R
CW = 3840
NFULL = 13
SPLIT = NFULL * CW
TAIL0 = 2 * NFULL * CW
TAILW = VOCAB - TAIL0
TPAD = 256
NEG_INF = float("-inf")

_GATHER_DNUMS = lax.GatherDimensionNumbers(
    offset_dims=(), collapsed_slice_dims=(0,), start_index_map=(0,))


def _shuf(x, perm):
    return lax.gather(
        x, perm[:, None], _GATHER_DNUMS, (1,),
        mode=lax.GatherScatterMode.PROMISE_IN_BOUNDS)


_mesh = plsc.VectorSubcoreMesh(core_axis_name="c", subcore_axis_name="s")


@functools.partial(
    pl.kernel,
    out_type=jax.ShapeDtypeStruct((NG * L,), jnp.int32),
    mesh=_mesh,
    scratch_types=[
        pltpu.VMEM((R, CW), jnp.float32),
        pltpu.VMEM((R, CW), jnp.float32),
        pltpu.VMEM((R, TPAD), jnp.float32),
        pltpu.VMEM((L,), jnp.int32),
        pltpu.VMEM((L,), jnp.float32),
        pltpu.VMEM((L,), jnp.int32),
        pltpu.VMEM_SHARED((NS * L,), jnp.float32),
        pltpu.VMEM_SHARED((NS * L,), jnp.int32),
        pltpu.SemaphoreType.DMA,
        pltpu.SemaphoreType.DMA,
        pltpu.SemaphoreType.DMA,
    ],
)
def _argmax_sc(logits_hbm, tail_hbm, out_hbm, buf0, buf1, tbuf, res_ref,
               mval_ref, midx_ref, sval, sidx, sem0, sem1, semt):
    cid = lax.axis_index("c")
    sid = lax.axis_index("s")
    h = sid % 2
    g = cid * (NS // 2) + sid // 2
    row0 = g * R
    col_base = h * SPLIT

    bufs = (buf0, buf1, tbuf)
    sems = (sem0, sem1, semt)
    blocks = [(c * CW, CW, c % 2) for c in range(NFULL)]
    blocks.append((None, TPAD, 2))

    def issue(t):
        off, w, slot = blocks[t]
        if off is None:
            @pl.when(h == 0)
            def _():
                pltpu.async_copy(
                    logits_hbm.at[pl.ds(row0, R), pl.ds(0, TPAD)],
                    tbuf, semt)

            @pl.when(h == 1)
            def _():
                pltpu.async_copy(tail_hbm.at[pl.ds(row0, R)], tbuf, semt)

            return pltpu.make_async_copy(
                logits_hbm.at[pl.ds(row0, R), pl.ds(0, TPAD)], tbuf, semt)
        return pltpu.async_copy(
            logits_hbm.at[pl.ds(row0, R), pl.ds(col_base + off, w)],
            bufs[slot], sems[slot])

    lane = lax.iota(jnp.int32, L)
    bvs = [jnp.full((L,), NEG_INF, jnp.float32) for _ in range(R)]
    bps = [lane for _ in range(R)]

    handles = [None] * len(blocks)
    handles[0] = issue(0)
    for t, (off, w, slot) in enumerate(blocks):
        if t + 1 < len(blocks):
            handles[t + 1] = issue(t + 1)
        handles[t].wait()
        buf = bufs[slot]
        col0 = h * TAIL0 if off is None else col_base + off

        def body(i, carry, buf=buf, col0=col0):
            bvs, bps = carry
            pcol = (col0 + i * L) + lane
            nbvs, nbps = [], []
            for r in range(R):
                v = buf[r, pl.ds(i * L, L)]
                m = v > bvs[r]
                nbvs.append(jnp.where(m, v, bvs[r]))
                nbps.append(jnp.where(m, pcol, bps[r]))
            return nbvs, nbps

        bvs, bps = plsc.parallel_loop(
            0, w // L, unroll=2, carry=(bvs, bps))(body)

    mval = jnp.full((L,), NEG_INF, jnp.float32)
    midx = jnp.zeros((L,), jnp.int32)
    for r in range(R):
        bv, bp = bvs[r], bps[r]
        for s in (8, 4, 2, 1):
            perm = lane ^ s
            ov = _shuf(bv, perm)
            op = _shuf(bp, perm)
            take = (ov > bv) | ((ov == bv) & (op < bp))
            bv = jnp.where(take, ov, bv)
            bp = jnp.where(take, op, bp)
        mval = jnp.where(lane == r, bv, mval)
        midx = jnp.where(lane == r, bp, midx)

    mval_ref[...] = mval
    midx_ref[...] = midx
    pltpu.sync_copy(mval_ref, sval.at[pl.ds(sid * L, L)])
    pltpu.sync_copy(midx_ref, sidx.at[pl.ds(sid * L, L)])
    plsc.subcore_barrier()

    @pl.when(h == 0)
    def _():
        pltpu.sync_copy(sval.at[pl.ds((sid + 1) * L, L)], mval_ref)
        pltpu.sync_copy(sidx.at[pl.ds((sid + 1) * L, L)], midx_ref)
        take = mval_ref[...] > mval
        res_ref[...] = jnp.where(take, midx_ref[...], midx)
        pltpu.sync_copy(res_ref, out_hbm.at[pl.ds(g * L, L)])


def kernel(logits):
    tail = jnp.pad(
        logits[:, TAIL0:], ((0, 0), (0, TPAD - TAILW)),
        constant_values=NEG_INF)
    out = _argmax_sc(logits, tail)
    return out.reshape(NG, L)[:, :R].reshape(BATCH)


# speedup vs baseline: 1.9142x; 1.9142x over previous
import functools

import jax
import jax.numpy as jnp
from jax import lax
from jax.experimental import pallas as pl
from jax.experimental.pallas import tpu as pltpu
from jax.experimental.pallas import tpu_sc as plsc

BATCH = 128
VOCAB = 100000
NC = 2
NS = 16
L = 16
NW = NC * NS
NB = BATCH // L
VW = 392
NCHK = 8
STRIP = NCHK * VW
LAST0 = VOCAB - STRIP
NEG_INF = float("-inf")

_mesh = plsc.VectorSubcoreMesh(core_axis_name="c", subcore_axis_name="s")


@functools.partial(
    pl.kernel,
    out_type=(
        jax.ShapeDtypeStruct((NC * BATCH,), jnp.int32),
        jax.ShapeDtypeStruct((NC * BATCH,), jnp.float32),
    ),
    mesh=_mesh,
    scratch_types=[
        pltpu.VMEM((VW, BATCH), jnp.float32),
        pltpu.VMEM((VW, BATCH), jnp.float32),
        pltpu.VMEM((BATCH,), jnp.float32),
        pltpu.VMEM((BATCH,), jnp.int32),
        pltpu.VMEM((NS * L,), jnp.float32),
        pltpu.VMEM((NS * L,), jnp.int32),
        pltpu.VMEM((L,), jnp.float32),
        pltpu.VMEM((L,), jnp.int32),
        pltpu.VMEM_SHARED((NS * BATCH,), jnp.float32),
        pltpu.VMEM_SHARED((NS * BATCH,), jnp.int32),
        pltpu.SemaphoreType.DMA,
        pltpu.SemaphoreType.DMA,
    ],
)
def _argmax_sc(xt_hbm, idx_hbm, val_hbm, buf0, buf1, stv, sti, gv, gi,
               rv, ri, sval, sidx, sem0, sem1):
    cid = lax.axis_index("c")
    sid = lax.axis_index("s")
    wid = sid * NC + cid
    start = pl.multiple_of(lax.min((wid * 3125) & ~7, LAST0), 8)

    bufs = (buf0, buf1)
    sems = (sem0, sem1)

    def issue(t):
        return pltpu.async_copy(
            xt_hbm.at[pl.ds(start + t * VW, VW)], bufs[t % 2], sems[t % 2])

    lane = lax.iota(jnp.int32, L)
    bvs = [jnp.full((L,), NEG_INF, jnp.float32) for _ in range(NB)]
    bps = [jnp.zeros((L,), jnp.int32) for _ in range(NB)]

    handles = [None] * NCHK
    handles[0] = issue(0)
    for t in range(NCHK):
        if t + 1 < NCHK:
            handles[t + 1] = issue(t + 1)
        handles[t].wait()
        buf = bufs[t % 2]
        base = start + t * VW

        def body(i, carry, buf=buf, base=base):
            bvs, bps = carry
            pos = jnp.full((L,), base + i, jnp.int32)
            nbvs, nbps = [], []
            for u in range(NB):
                v = buf[i, pl.ds(u * L, L)]
                m = v > bvs[u]
                nbvs.append(jnp.where(m, v, bvs[u]))
                nbps.append(jnp.where(m, pos, bps[u]))
            return nbvs, nbps

        bvs, bps = plsc.parallel_loop(
            0, VW, unroll=2, carry=(bvs, bps))(body)

    for u in range(NB):
        stv[pl.ds(u * L, L)] = bvs[u]
        sti[pl.ds(u * L, L)] = bps[u]
    pltpu.sync_copy(stv, sval.at[pl.ds(sid * BATCH, BATCH)])
    pltpu.sync_copy(sti, sidx.at[pl.ds(sid * BATCH, BATCH)])
    plsc.subcore_barrier()

    @pl.when(sid < NB)
    def _():
        for j in range(NS):
            pltpu.sync_copy(
                sval.at[pl.ds(j * BATCH + sid * L, L)],
                gv.at[pl.ds(j * L, L)])
            pltpu.sync_copy(
                sidx.at[pl.ds(j * BATCH + sid * L, L)],
                gi.at[pl.ds(j * L, L)])
        av = gv[pl.ds(0, L)]
        ai = gi[pl.ds(0, L)]
        for j in range(1, NS):
            ov = gv[pl.ds(j * L, L)]
            oi = gi[pl.ds(j * L, L)]
            take = (ov > av) | ((ov == av) & (oi < ai))
            av = jnp.where(take, ov, av)
            ai = jnp.where(take, oi, ai)
        rv[...] = av
        ri[...] = ai
        pltpu.sync_copy(ri, idx_hbm.at[pl.ds(cid * BATCH + sid * L, L)])
        pltpu.sync_copy(rv, val_hbm.at[pl.ds(cid * BATCH + sid * L, L)])


def kernel(logits):
    idx, val = _argmax_sc(logits.T)
    vi = val.reshape(NC, BATCH)
    ii = idx.reshape(NC, BATCH)
    take = (vi[1] > vi[0]) | ((vi[1] == vi[0]) & (ii[1] < ii[0]))
    return jnp.where(take, ii[1], ii[0])
